# baseline (speedup 1.0000x reference)
import functools

import jax
import jax.numpy as jnp
from jax import lax
from jax.experimental import pallas as pl
from jax.experimental.pallas import tpu as pltpu
from jax.experimental.pallas import tpu_sc as plsc

B_, L_, C_ = 2, 4096, 1024
H_, K_ = 16, 16
D_ = C_ // H_
M_ = B_ * L_
J_ = K_ + 1
JPAD = 32
MAX_OFFSET = 64
HALF_K = K_ // 2
TB = 256
TSC = 128
HALO = MAX_OFFSET + HALF_K
SLAB = TSC + 2 * HALO
NBLK = L_ // TSC
NTEC = 32
NTASK = B_ * H_ * NBLK
VS = D_ + 1
US = JPAD + 1


def _stage1_body(x_ref, wkv_ref, wwo_ref, b_ref, g_ref, v_ref, u_ref):
    i = pl.program_id(0)
    xb = x_ref[...]
    pkv = jnp.dot(xb, wkv_ref[...], preferred_element_type=jnp.float32)
    pwo = jnp.dot(xb, wwo_ref[...], preferred_element_type=jnp.float32)
    vv = pkv[:, :C_] + b_ref[0, :C_]
    kl = pkv[:, C_:] + b_ref[0, C_:C_ + H_ * K_]
    wl = pwo[:, :H_] + b_ref[0, C_ + H_ * K_:C_ + H_ * K_ + H_]
    ol = pwo[:, H_:] + b_ref[0, C_ + H_ * K_ + H_:]
    v_ref[...] = vv

    def rms(z):
        return jnp.sqrt(jnp.mean(z * z, axis=-1, keepdims=True))

    kn = kl / (rms(kl) + 1e-6) * g_ref[0, :H_ * K_]
    wn = wl / (rms(wl) + 1e-6) * g_ref[0, H_ * K_:H_ * K_ + H_]
    on = ol / (rms(ol) + 1e-6) * g_ref[0, H_ * K_ + H_:]

    half = 2.0 + 6.0 * jax.nn.sigmoid(wn)
    off = jnp.tanh(on) * float(MAX_OFFSET)
    kw = kn * jax.nn.sigmoid(kn)
    kpad = jnp.concatenate(
        [kw.reshape(TB, H_, K_), jnp.zeros((TB, H_, JPAD - K_), jnp.float32)],
        axis=2)

    jlane = lax.broadcasted_iota(jnp.int32, (TB, H_, JPAD), 2)
    relabs = jnp.abs(jlane - HALF_K).astype(jnp.float32)
    wkp = kpad * jax.nn.sigmoid(half[:, :, None] - relabs)

    o0f = jnp.floor(off)
    frac = off - o0f
    o0 = o0f.astype(jnp.int32)

    u = ((1.0 - frac)[:, :, None] * wkp
         + frac[:, :, None] * jnp.roll(wkp, 1, axis=2))

    l0 = (i * TB) % L_
    lt = l0 + lax.broadcasted_iota(jnp.int32, (TB, H_), 0)
    base = lt + o0 - HALF_K
    p = base[:, :, None] + jlane
    u = jnp.where((p >= 0) & (p < L_), u, 0.0)
    base_f = lax.bitcast_convert_type(base, jnp.float32)[:, :, None]
    u_ref[...] = jnp.where(jlane == JPAD - 1, base_f, u)


def _stage3_body(h_ref, w_ref, o_ref):
    y = jnp.dot(h_ref[...], w_ref[...], preferred_element_type=jnp.float32)
    o_ref[...] = y * jax.nn.sigmoid(y)


def _conv_sc_body(v_hbm, u_hbm, out_hbm, slab0, slab1, ub, ob, sem0, sem1):
    cid = lax.axis_index("c")
    sid = lax.axis_index("s")
    wid = sid * 2 + cid
    NT = NTASK // NTEC
    bufs = ((slab0, sem0), (slab1, sem1))

    def decode(ti):
        tid = ti * NTEC + wid
        b = tid // (H_ * NBLK)
        rem = tid % (H_ * NBLK)
        h = rem // NBLK
        blk = rem % NBLK
        tok0 = pl.multiple_of(blk * TSC, TSC)
        start = pl.multiple_of(jnp.clip(tok0 - HALO, 0, L_ - SLAB), 8)
        return b, h, tok0, start

    def in_copy(ti, buf):
        slab, sem = buf
        b, h, tok0, start = decode(ti)
        return pltpu.make_async_copy(
            v_hbm.at[b, h, pl.ds(start, SLAB), :], slab, sem)

    def start_in(ti, buf):
        in_copy(ti, buf).start()

    def run_task(ti, buf):
        slab, sem = buf
        in_copy(ti, buf).wait()
        b, h, tok0, start = decode(ti)
        pltpu.sync_copy(u_hbm.at[b, h, pl.ds(tok0, TSC), :], ub)
        lane = lax.iota(jnp.int32, 16)

        @plsc.parallel_loop(0, TSC)
        def tok_body(t):
            tv = jnp.broadcast_to(t, (16,))
            braw = plsc.load_gather(
                ub, [tv, jnp.broadcast_to(jnp.int32(JPAD - 1), (16,))])
            bbits = plsc.bitcast(braw, jnp.int32)
            base_i = jnp.sum(jnp.where(lane == 0, bbits, 0)) - start
            accs = [jnp.zeros((16,), jnp.float32) for _ in range(4)]
            for j in range(J_):
                wj = plsc.load_gather(
                    ub, [tv, jnp.broadcast_to(jnp.int32(j), (16,))])
                r = jnp.clip(base_i + j, 0, SLAB - 1)
                for q in range(4):
                    accs[q] = accs[q] + wj * slab[r, pl.ds(q * 16, 16)]
            for q in range(4):
                ob[t, pl.ds(q * 16, 16)] = accs[q]

        pltpu.sync_copy(ob, out_hbm.at[b, h, pl.ds(tok0, TSC), :])

    start_in(0, bufs[0])

    def pair_body(i, carry):
        ti = i * 2
        start_in(ti + 1, bufs[1])
        run_task(ti, bufs[0])

        @pl.when(ti + 2 < NT)
        def _():
            start_in(ti + 2, bufs[0])

        run_task(ti + 1, bufs[1])
        return carry

    lax.fori_loop(0, NT // 2, pair_body, 0)


def kernel(x, window_w, window_b, window_gamma, offset_w, offset_b,
           offset_gamma, kernel_w, kernel_b, kernel_gamma, v_w, v_b, out_w):
    f32 = jnp.float32
    xf = x.reshape(M_, C_)
    wkv = jnp.concatenate([v_w, kernel_w], axis=0).T
    wwo = jnp.concatenate([window_w, offset_w], axis=0).T
    ball = jnp.concatenate([v_b, kernel_b, window_b, offset_b])[None, :]
    gall = jnp.concatenate([kernel_gamma, window_gamma, offset_gamma])[None, :]

    v, u = pl.pallas_call(
        _stage1_body,
        grid=(M_ // TB,),
        in_specs=[
            pl.BlockSpec((TB, C_), lambda i: (i, 0)),
            pl.BlockSpec((C_, C_ + H_ * K_), lambda i: (0, 0)),
            pl.BlockSpec((C_, 2 * H_), lambda i: (0, 0)),
            pl.BlockSpec((1, C_ + H_ * K_ + 2 * H_), lambda i: (0, 0)),
            pl.BlockSpec((1, H_ * K_ + 2 * H_), lambda i: (0, 0)),
        ],
        out_specs=[
            pl.BlockSpec((TB, C_), lambda i: (i, 0)),
            pl.BlockSpec((TB, H_, JPAD), lambda i: (i, 0, 0)),
        ],
        out_shape=[
            jax.ShapeDtypeStruct((M_, C_), f32),
            jax.ShapeDtypeStruct((M_, H_, JPAD), f32),
        ],
    )(xf, wkv, wwo, ball, gall)

    vt = v.reshape(B_, L_, H_, D_).transpose(0, 2, 1, 3)
    ut = u.reshape(B_, L_, H_, JPAD).transpose(0, 2, 1, 3)

    mesh = plsc.VectorSubcoreMesh(core_axis_name="c", subcore_axis_name="s")
    hid_t = pl.kernel(
        _conv_sc_body,
        out_type=jax.ShapeDtypeStruct((B_, H_, L_, D_), f32),
        mesh=mesh,
        compiler_params=pltpu.CompilerParams(needs_layout_passes=False),
        scratch_types=[
            pltpu.VMEM((SLAB, VS), f32),
            pltpu.VMEM((SLAB, VS), f32),
            pltpu.VMEM((TSC, US), f32),
            pltpu.VMEM((TSC, D_), f32),
            pltpu.SemaphoreType.DMA,
            pltpu.SemaphoreType.DMA,
        ],
    )(
        jnp.pad(vt, ((0, 0), (0, 0), (0, 0), (0, 1))),
        jnp.pad(ut, ((0, 0), (0, 0), (0, 0), (0, 1))),
    )

    hid = hid_t.transpose(0, 2, 1, 3).reshape(M_, C_)

    out = pl.pallas_call(
        _stage3_body,
        grid=(M_ // TB,),
        in_specs=[
            pl.BlockSpec((TB, C_), lambda i: (i, 0)),
            pl.BlockSpec((C_, C_), lambda i: (0, 0)),
        ],
        out_specs=pl.BlockSpec((TB, C_), lambda i: (i, 0)),
        out_shape=jax.ShapeDtypeStruct((M_, C_), f32),
    )(hid, out_w.T)

    return out.reshape(B_, L_, C_)
